# Initial kernel scaffold; baseline (speedup 1.0000x reference)
import jax
import jax.numpy as jnp
from jax.experimental import pallas as pl


def kernel(x, edge_index, W_enc, b_enc, W1, b1, W2, b2, W_cls, b_cls):
    raise NotImplementedError("write your pallas kernel here")


# speedup vs baseline: 6.8217x; 6.8217x over previous
import functools

import jax
import jax.numpy as jnp
from jax import lax
from jax.experimental import pallas as pl
from jax.experimental.pallas import tpu as pltpu
from jax.experimental.pallas import tpu_sc as plsc

N = 50000
E = 800000
DIN = 21
H = 128
V = 4

NP = 50176
R = 1024
GRID = NP // R
NCB = 4
CBW = H // NCB
TILE_ROWS = NP // 16
STAGE_ROWS = TILE_ROWS // 4
EK = 128
ECHUNKS = E // EK
ECHUNKS_HALF = ECHUNKS // 2
S2_ITERS = (ECHUNKS + 15) // 16
S0_ITERS = (ECHUNKS_HALF + 15) // 16


def _sc_mesh():
    return plsc.VectorSubcoreMesh(core_axis_name="c", subcore_axis_name="s")


def _s0_body(ei_hbm, deg_hbm, idx_v, ones_v, zbuf_v, deg_sp):
    c = lax.axis_index("c")
    s = lax.axis_index("s")

    def fill(buf, n, val):
        def body(i, _):
            buf[pl.ds(i * 16, 16)] = jnp.full((16,), val, jnp.float32)
            return _
        lax.fori_loop(0, n // 16, body, None)

    fill(ones_v, EK, 1.0)
    fill(zbuf_v, TILE_ROWS, 0.0)
    pltpu.sync_copy(zbuf_v, deg_sp.at[pl.ds(s * TILE_ROWS, TILE_ROWS)])
    plsc.subcore_barrier()

    def edge_chunk(j, _):
        chunk = s + 16 * j

        @pl.when(chunk < ECHUNKS_HALF)
        def _():
            off = c * (E // 2) + chunk * EK
            pltpu.sync_copy(ei_hbm.at[1, pl.ds(off, EK)], idx_v)
            pltpu.sync_copy(ones_v, deg_sp.at[idx_v], add=True)

        return _

    lax.fori_loop(0, S0_ITERS, edge_chunk, None)
    plsc.subcore_barrier()
    sl = pl.ds(s * TILE_ROWS, TILE_ROWS)
    pltpu.sync_copy(deg_sp.at[sl], zbuf_v)
    pltpu.sync_copy(zbuf_v, deg_hbm.at[pl.ds(c * NP + s * TILE_ROWS, TILE_ROWS)])


def _s0(edge_index):
    return pl.kernel(
        _s0_body,
        out_type=jax.ShapeDtypeStruct((2 * NP,), jnp.float32),
        mesh=_sc_mesh(),
        scratch_types=[
            pltpu.VMEM((EK,), jnp.int32),
            pltpu.VMEM((EK,), jnp.float32),
            pltpu.VMEM((TILE_ROWS,), jnp.float32),
            pltpu.VMEM_SHARED((NP,), jnp.float32),
        ],
    )(edge_index)


def _s2_body(ei_hbm, g_hbm, acc_hbm, idx_s, idx_d, msg_v, stage_v, acc_sp):
    c = lax.axis_index("c")
    s = lax.axis_index("s")
    base = s * TILE_ROWS

    for p in range(2):
        cb = 2 * c + p
        for q in range(4):
            sl = pl.ds(base + q * STAGE_ROWS, STAGE_ROWS)
            pltpu.sync_copy(g_hbm.at[cb, sl, :], stage_v)
            pltpu.sync_copy(stage_v, acc_sp.at[sl, :])
        plsc.subcore_barrier()

        def edge_chunk(j, _):
            chunk = s + 16 * j

            @pl.when(chunk < ECHUNKS)
            def _():
                off = chunk * EK
                pltpu.sync_copy(ei_hbm.at[0, pl.ds(off, EK)], idx_s)
                pltpu.sync_copy(ei_hbm.at[1, pl.ds(off, EK)], idx_d)
                pltpu.sync_copy(g_hbm.at[cb].at[idx_s], msg_v)
                pltpu.sync_copy(msg_v, acc_sp.at[idx_d], add=True)

            return _

        lax.fori_loop(0, S2_ITERS, edge_chunk, None)
        plsc.subcore_barrier()
        for q in range(4):
            sl = pl.ds(base + q * STAGE_ROWS, STAGE_ROWS)
            pltpu.sync_copy(acc_sp.at[sl, :], stage_v)
            pltpu.sync_copy(stage_v, acc_hbm.at[cb, sl, :])
        plsc.subcore_barrier()


def _s2(edge_index, g):
    return pl.kernel(
        _s2_body,
        out_type=jax.ShapeDtypeStruct((NCB, NP, CBW), jnp.float32),
        mesh=_sc_mesh(),
        scratch_types=[
            pltpu.VMEM((EK,), jnp.int32),
            pltpu.VMEM((EK,), jnp.int32),
            pltpu.VMEM((EK, CBW), jnp.float32),
            pltpu.VMEM((STAGE_ROWS, CBW), jnp.float32),
            pltpu.VMEM_SHARED((NP, CBW), jnp.float32),
        ],
        compiler_params=pltpu.CompilerParams(use_tc_tiling_on_sc=False),
    )(edge_index, g)


_PREC = lax.Precision.HIGHEST


def _k1_body(x_ref, deg_ref, we_ref, be_ref, w1_ref, g_ref, dinv_ref):
    deg = deg_ref[0, :] + deg_ref[1, :] + 1.0
    dinv = lax.rsqrt(deg)
    h = jnp.maximum(
        jnp.dot(x_ref[...], we_ref[...], precision=_PREC,
                preferred_element_type=jnp.float32) + be_ref[...], 0.0)
    g = jnp.dot(h * dinv[:, None], w1_ref[...], precision=_PREC,
                preferred_element_type=jnp.float32)
    dinv_ref[...] = dinv
    for cb in range(NCB):
        g_ref[cb] = g[:, cb * CBW:(cb + 1) * CBW]


def _k1(x_pad, deg2, W_enc, b_enc2d, W1):
    return pl.pallas_call(
        _k1_body,
        grid=(GRID,),
        in_specs=[
            pl.BlockSpec((R, DIN), lambda i: (i, 0)),
            pl.BlockSpec((2, R), lambda i: (0, i)),
            pl.BlockSpec((DIN, H), lambda i: (0, 0)),
            pl.BlockSpec((1, H), lambda i: (0, 0)),
            pl.BlockSpec((H, H), lambda i: (0, 0)),
        ],
        out_specs=[
            pl.BlockSpec((NCB, R, CBW), lambda i: (0, i, 0)),
            pl.BlockSpec((R,), lambda i: (i,)),
        ],
        out_shape=[
            jax.ShapeDtypeStruct((NCB, NP, CBW), jnp.float32),
            jax.ShapeDtypeStruct((NP,), jnp.float32),
        ],
    )(x_pad, deg2, W_enc, b_enc2d, W1)


def _k2_body(acc_ref, dinv_ref, b_ref, w_ref, g_ref):
    acc = jnp.concatenate([acc_ref[cb] for cb in range(NCB)], axis=1)
    dinv = dinv_ref[...]
    h = jnp.maximum(dinv[:, None] * acc + b_ref[...], 0.0)
    g = jnp.dot(h * dinv[:, None], w_ref[...], precision=_PREC,
                preferred_element_type=jnp.float32)
    for cb in range(NCB):
        g_ref[cb] = g[:, cb * CBW:(cb + 1) * CBW]


def _k2(acc, dinv, b2d, W):
    return pl.pallas_call(
        _k2_body,
        grid=(GRID,),
        in_specs=[
            pl.BlockSpec((NCB, R, CBW), lambda i: (0, i, 0)),
            pl.BlockSpec((R,), lambda i: (i,)),
            pl.BlockSpec((1, H), lambda i: (0, 0)),
            pl.BlockSpec((H, H), lambda i: (0, 0)),
        ],
        out_specs=pl.BlockSpec((NCB, R, CBW), lambda i: (0, i, 0)),
        out_shape=jax.ShapeDtypeStruct((NCB, NP, CBW), jnp.float32),
    )(acc, dinv, b2d, W)


def _k3_body(acc_ref, dinv_ref, b_ref, wc_ref, bc_ref, out_ref):
    acc = jnp.concatenate([acc_ref[cb] for cb in range(NCB)], axis=1)
    dinv = dinv_ref[...]
    h = jnp.maximum(dinv[:, None] * acc + b_ref[...], 0.0)
    out_ref[...] = jnp.dot(h, wc_ref[...], precision=_PREC,
                           preferred_element_type=jnp.float32) + bc_ref[...]


def _k3(acc, dinv, b2d, W_cls, b_cls2d):
    return pl.pallas_call(
        _k3_body,
        grid=(GRID,),
        in_specs=[
            pl.BlockSpec((NCB, R, CBW), lambda i: (0, i, 0)),
            pl.BlockSpec((R,), lambda i: (i,)),
            pl.BlockSpec((1, H), lambda i: (0, 0)),
            pl.BlockSpec((H, V), lambda i: (0, 0)),
            pl.BlockSpec((1, V), lambda i: (0, 0)),
        ],
        out_specs=pl.BlockSpec((R, V), lambda i: (i, 0)),
        out_shape=jax.ShapeDtypeStruct((NP, V), jnp.float32),
    )(acc, dinv, b2d, W_cls, b_cls2d)


def kernel(x, edge_index, W_enc, b_enc, W1, b1, W2, b2, W_cls, b_cls):
    x_pad = jnp.pad(x, ((0, NP - N), (0, 0)))
    deg2 = _s0(edge_index).reshape(2, NP)
    g1, dinv = _k1(x_pad, deg2, W_enc, b_enc.reshape(1, H), W1)
    acc1 = _s2(edge_index, g1)
    g2 = _k2(acc1, dinv, b1.reshape(1, H), W2)
    acc2 = _s2(edge_index, g2)
    out = _k3(acc2, dinv, b2.reshape(1, H), W_cls, b_cls.reshape(1, V))
    return out[:N]


# speedup vs baseline: 13.2230x; 1.9384x over previous
import functools

import jax
import jax.numpy as jnp
from jax import lax
from jax.experimental import pallas as pl
from jax.experimental.pallas import tpu as pltpu
from jax.experimental.pallas import tpu_sc as plsc

N = 50000
E = 800000
DIN = 21
H = 128
V = 4

NP = 50176
R = 1024
GRID = NP // R
NCB = 8
CBW = H // NCB
TILE_ROWS = NP // 16
STAGE_ROWS = TILE_ROWS // 4
EK = 128
EPAD = 819200
ECHUNKS = EPAD // EK
TILE_CHUNKS = ECHUNKS // 16
G = 8
GROUPS = TILE_CHUNKS // G
S0_TILE_CHUNKS = ECHUNKS // 2 // 16


def _sc_mesh():
    return plsc.VectorSubcoreMesh(core_axis_name="c", subcore_axis_name="s")


def _s0_body(ei_hbm, deg_hbm, idx_v, ones_v, zbuf_v, deg_sp):
    c = lax.axis_index("c")
    s = lax.axis_index("s")

    def fill(buf, n, val):
        def body(i, _):
            buf[pl.ds(i * 16, 16)] = jnp.full((16,), val, jnp.float32)
            return _
        lax.fori_loop(0, n // 16, body, None)

    fill(ones_v, EK, 1.0)
    fill(zbuf_v, TILE_ROWS, 0.0)
    pltpu.sync_copy(zbuf_v, deg_sp.at[pl.ds(s * TILE_ROWS, TILE_ROWS)])
    plsc.subcore_barrier()

    row0 = (c * 16 + s) * S0_TILE_CHUNKS

    def edge_chunk(j, _):
        pltpu.sync_copy(ei_hbm.at[row0 + j, 1, :], idx_v)
        pltpu.sync_copy(ones_v, deg_sp.at[idx_v], add=True)
        return _

    lax.fori_loop(0, S0_TILE_CHUNKS, edge_chunk, None)
    plsc.subcore_barrier()
    sl = pl.ds(s * TILE_ROWS, TILE_ROWS)
    pltpu.sync_copy(deg_sp.at[sl], zbuf_v)
    pltpu.sync_copy(zbuf_v, deg_hbm.at[pl.ds(c * NP + s * TILE_ROWS, TILE_ROWS)])


def _s0(edge_index):
    return pl.kernel(
        _s0_body,
        out_type=jax.ShapeDtypeStruct((2 * NP,), jnp.float32),
        mesh=_sc_mesh(),
        scratch_types=[
            pltpu.VMEM((EK,), jnp.int32),
            pltpu.VMEM((EK,), jnp.float32),
            pltpu.VMEM((TILE_ROWS,), jnp.float32),
            pltpu.VMEM_SHARED((NP,), jnp.float32),
        ],
        compiler_params=pltpu.CompilerParams(use_tc_tiling_on_sc=False),
    )(edge_index)


def _s2_body(ei_hbm, g_hbm, acc_hbm, ibuf, msg_v, acc_sp,
             sem_i, sem_g, sem_s0, sem_s1):
    c = lax.axis_index("c")
    s = lax.axis_index("s")
    base = s * TILE_ROWS
    row_base = s * TILE_CHUNKS
    sem_s = (sem_s0, sem_s1)

    def run_pass(p, _):
        cb = (NCB // 2) * c + p
        stage_v = msg_v.at[0, pl.ds(0, STAGE_ROWS), :]
        for q in range(4):
            sl = pl.ds(base + q * STAGE_ROWS, STAGE_ROWS)
            pltpu.sync_copy(g_hbm.at[cb, sl, :], stage_v)
            pltpu.sync_copy(stage_v, acc_sp.at[sl, :])

        def fire_idx(g):
            return pltpu.async_copy(
                ei_hbm.at[pl.ds(row_base + g * G, G), :, :],
                ibuf.at[g % 3], sem_i)

        def fire_gathers(g):
            ds_ = []
            for i in range(G):
                ds_.append(pltpu.async_copy(
                    g_hbm.at[cb].at[ibuf.at[g % 3, i, 0]],
                    msg_v.at[g % 2, pl.ds(i * EK, EK), :], sem_g))
            return ds_

        def fire_scatters(g):
            ds_ = []
            for i in range(G):
                ds_.append(pltpu.async_copy(
                    msg_v.at[g % 2, pl.ds(i * EK, EK), :],
                    acc_sp.at[ibuf.at[g % 3, i, 1]], sem_s[g % 2],
                    add=True))
            return ds_

        idx_d = fire_idx(0)
        plsc.subcore_barrier()

        scat = {}
        for g in range(GROUPS):
            idx_d.wait()
            if g >= 2:
                for d in scat.pop(g - 2):
                    d.wait()
            gat = fire_gathers(g)
            if g + 1 < GROUPS:
                idx_d = fire_idx(g + 1)
            for d in gat:
                d.wait()
            scat[g] = fire_scatters(g)
        for g_ in sorted(scat):
            for d in scat.pop(g_):
                d.wait()

        plsc.subcore_barrier()
        for q in range(4):
            sl = pl.ds(base + q * STAGE_ROWS, STAGE_ROWS)
            pltpu.sync_copy(acc_sp.at[sl, :], stage_v)
            pltpu.sync_copy(stage_v, acc_hbm.at[cb, sl, :])
        plsc.subcore_barrier()
        return _

    lax.fori_loop(0, NCB // 2, run_pass, None)


def _s2(edge_index, g):
    return pl.kernel(
        _s2_body,
        out_type=jax.ShapeDtypeStruct((NCB, NP, CBW), jnp.float32),
        mesh=_sc_mesh(),
        scratch_types=[
            pltpu.VMEM((3, G, 2, EK), jnp.int32),
            pltpu.VMEM((2, G * EK, CBW), jnp.float32),
            pltpu.VMEM_SHARED((NP, CBW), jnp.float32),
            pltpu.SemaphoreType.DMA,
            pltpu.SemaphoreType.DMA,
            pltpu.SemaphoreType.DMA,
            pltpu.SemaphoreType.DMA,
        ],
        compiler_params=pltpu.CompilerParams(use_tc_tiling_on_sc=False),
    )(edge_index, g)


_PREC = lax.Precision.HIGHEST


def _k1_body(x_ref, deg_ref, we_ref, be_ref, w1_ref, g_ref, dinv_ref):
    deg = deg_ref[0, :] + deg_ref[1, :] + 1.0
    dinv = lax.rsqrt(deg)
    h = jnp.maximum(
        jnp.dot(x_ref[...], we_ref[...], precision=_PREC,
                preferred_element_type=jnp.float32) + be_ref[...], 0.0)
    g = jnp.dot(h * dinv[:, None], w1_ref[...], precision=_PREC,
                preferred_element_type=jnp.float32)
    dinv_ref[...] = dinv
    for cb in range(NCB):
        g_ref[cb] = g[:, cb * CBW:(cb + 1) * CBW]


def _k1(x_pad, deg2, W_enc, b_enc2d, W1):
    return pl.pallas_call(
        _k1_body,
        grid=(GRID,),
        in_specs=[
            pl.BlockSpec((R, DIN), lambda i: (i, 0)),
            pl.BlockSpec((2, R), lambda i: (0, i)),
            pl.BlockSpec((DIN, H), lambda i: (0, 0)),
            pl.BlockSpec((1, H), lambda i: (0, 0)),
            pl.BlockSpec((H, H), lambda i: (0, 0)),
        ],
        out_specs=[
            pl.BlockSpec((NCB, R, CBW), lambda i: (0, i, 0)),
            pl.BlockSpec((R,), lambda i: (i,)),
        ],
        out_shape=[
            jax.ShapeDtypeStruct((NCB, NP, CBW), jnp.float32),
            jax.ShapeDtypeStruct((NP,), jnp.float32),
        ],
    )(x_pad, deg2, W_enc, b_enc2d, W1)


def _k2_body(acc_ref, dinv_ref, b_ref, w_ref, g_ref):
    acc = jnp.concatenate([acc_ref[cb] for cb in range(NCB)], axis=1)
    dinv = dinv_ref[...]
    h = jnp.maximum(dinv[:, None] * acc + b_ref[...], 0.0)
    g = jnp.dot(h * dinv[:, None], w_ref[...], precision=_PREC,
                preferred_element_type=jnp.float32)
    for cb in range(NCB):
        g_ref[cb] = g[:, cb * CBW:(cb + 1) * CBW]


def _k2(acc, dinv, b2d, W):
    return pl.pallas_call(
        _k2_body,
        grid=(GRID,),
        in_specs=[
            pl.BlockSpec((NCB, R, CBW), lambda i: (0, i, 0)),
            pl.BlockSpec((R,), lambda i: (i,)),
            pl.BlockSpec((1, H), lambda i: (0, 0)),
            pl.BlockSpec((H, H), lambda i: (0, 0)),
        ],
        out_specs=pl.BlockSpec((NCB, R, CBW), lambda i: (0, i, 0)),
        out_shape=jax.ShapeDtypeStruct((NCB, NP, CBW), jnp.float32),
    )(acc, dinv, b2d, W)


def _k3_body(acc_ref, dinv_ref, b_ref, wc_ref, bc_ref, out_ref):
    acc = jnp.concatenate([acc_ref[cb] for cb in range(NCB)], axis=1)
    dinv = dinv_ref[...]
    h = jnp.maximum(dinv[:, None] * acc + b_ref[...], 0.0)
    out_ref[...] = jnp.dot(h, wc_ref[...], precision=_PREC,
                           preferred_element_type=jnp.float32) + bc_ref[...]


def _k3(acc, dinv, b2d, W_cls, b_cls2d):
    return pl.pallas_call(
        _k3_body,
        grid=(GRID,),
        in_specs=[
            pl.BlockSpec((NCB, R, CBW), lambda i: (0, i, 0)),
            pl.BlockSpec((R,), lambda i: (i,)),
            pl.BlockSpec((1, H), lambda i: (0, 0)),
            pl.BlockSpec((H, V), lambda i: (0, 0)),
            pl.BlockSpec((1, V), lambda i: (0, 0)),
        ],
        out_specs=pl.BlockSpec((R, V), lambda i: (i, 0)),
        out_shape=jax.ShapeDtypeStruct((NP, V), jnp.float32),
    )(acc, dinv, b2d, W_cls, b_cls2d)


def kernel(x, edge_index, W_enc, b_enc, W1, b1, W2, b2, W_cls, b_cls):
    x_pad = jnp.pad(x, ((0, NP - N), (0, 0)))
    npad = EPAD - E
    pad_src = (jnp.arange(npad, dtype=jnp.int32) % N)
    pad_dst = N + (jnp.arange(npad, dtype=jnp.int32) % (NP - N))
    ei3 = (jnp.concatenate(
        [edge_index, jnp.stack([pad_src, pad_dst])], axis=1)
        .reshape(2, ECHUNKS, EK).transpose(1, 0, 2))
    deg2 = _s0(ei3).reshape(2, NP)
    g1, dinv = _k1(x_pad, deg2, W_enc, b_enc.reshape(1, H), W1)
    acc1 = _s2(ei3, g1)
    g2 = _k2(acc1, dinv, b1.reshape(1, H), W2)
    acc2 = _s2(ei3, g2)
    out = _k3(acc2, dinv, b2.reshape(1, H), W_cls, b_cls.reshape(1, V))
    return out[:N]


# speedup vs baseline: 13.6631x; 1.0333x over previous
import functools

import jax
import jax.numpy as jnp
from jax import lax
from jax.experimental import pallas as pl
from jax.experimental.pallas import tpu as pltpu
from jax.experimental.pallas import tpu_sc as plsc

N = 50000
E = 800000
DIN = 21
H = 128
V = 4

NP = 50176
R = 1024
GRID = NP // R
NCB = 8
CBW = H // NCB
TILE_ROWS = NP // 16
STAGE_ROWS = TILE_ROWS // 4
EK = 128
EPAD = 819200
ECHUNKS = EPAD // EK
TILE_CHUNKS = ECHUNKS // 16
G = 8
GROUPS = TILE_CHUNKS // G
S0_TILE_CHUNKS = ECHUNKS // 2 // 16


def _sc_mesh():
    return plsc.VectorSubcoreMesh(core_axis_name="c", subcore_axis_name="s")


def _s0_body(ei_hbm, deg_hbm, idx_v, ones_v, zbuf_v, deg_sp):
    c = lax.axis_index("c")
    s = lax.axis_index("s")

    def fill(buf, n, val):
        def body(i, _):
            buf[pl.ds(i * 16, 16)] = jnp.full((16,), val, jnp.float32)
            return _
        lax.fori_loop(0, n // 16, body, None)

    fill(ones_v, EK, 1.0)
    fill(zbuf_v, TILE_ROWS, 0.0)
    pltpu.sync_copy(zbuf_v, deg_sp.at[pl.ds(s * TILE_ROWS, TILE_ROWS)])
    plsc.subcore_barrier()

    row0 = (c * 16 + s) * S0_TILE_CHUNKS

    def edge_chunk(j, _):
        pltpu.sync_copy(ei_hbm.at[row0 + j, 1, :], idx_v)
        pltpu.sync_copy(ones_v, deg_sp.at[idx_v], add=True)
        return _

    lax.fori_loop(0, S0_TILE_CHUNKS, edge_chunk, None)
    plsc.subcore_barrier()
    sl = pl.ds(s * TILE_ROWS, TILE_ROWS)
    pltpu.sync_copy(deg_sp.at[sl], zbuf_v)
    pltpu.sync_copy(zbuf_v, deg_hbm.at[pl.ds(c * NP + s * TILE_ROWS, TILE_ROWS)])


def _s0(edge_index):
    return pl.kernel(
        _s0_body,
        out_type=jax.ShapeDtypeStruct((2 * NP,), jnp.float32),
        mesh=_sc_mesh(),
        scratch_types=[
            pltpu.VMEM((EK,), jnp.int32),
            pltpu.VMEM((EK,), jnp.float32),
            pltpu.VMEM((TILE_ROWS,), jnp.float32),
            pltpu.VMEM_SHARED((NP,), jnp.float32),
        ],
        compiler_params=pltpu.CompilerParams(use_tc_tiling_on_sc=False),
    )(edge_index)


def _s2_body(ei_hbm, g_hbm, acc_hbm, ibuf, msg_v, acc_sp,
             sem_i, sem_g, sem_s0, sem_s1):
    c = lax.axis_index("c")
    s = lax.axis_index("s")
    base = s * TILE_ROWS
    row_base = s * TILE_CHUNKS
    sem_s = (sem_s0, sem_s1)

    def run_pass(p, _):
        cb = (NCB // 2) * c + p
        stage_v = msg_v.at[0, pl.ds(0, STAGE_ROWS), :]
        for q in range(4):
            sl = pl.ds(base + q * STAGE_ROWS, STAGE_ROWS)
            pltpu.sync_copy(g_hbm.at[cb, sl, :], stage_v)
            pltpu.sync_copy(stage_v, acc_sp.at[sl, :])

        def fire_idx(g):
            return pltpu.async_copy(
                ei_hbm.at[pl.ds(row_base + g * G, G), :, :],
                ibuf.at[g % 3], sem_i)

        def fire_gathers(g):
            ds_ = []
            for i in range(G):
                ds_.append(pltpu.async_copy(
                    g_hbm.at[cb].at[ibuf.at[g % 3, i, 0]],
                    msg_v.at[g % 2, pl.ds(i * EK, EK), :], sem_g))
            return ds_

        def fire_scatters(g):
            ds_ = []
            for i in range(G):
                ds_.append(pltpu.async_copy(
                    msg_v.at[g % 2, pl.ds(i * EK, EK), :],
                    acc_sp.at[ibuf.at[g % 3, i, 1]], sem_s[g % 2],
                    add=True))
            return ds_

        idx_d = fire_idx(0)
        plsc.subcore_barrier()

        scat = {}
        for g in range(GROUPS):
            idx_d.wait()
            if g >= 2:
                for d in scat.pop(g - 2):
                    d.wait()
            gat = fire_gathers(g)
            if g + 1 < GROUPS:
                idx_d = fire_idx(g + 1)
            for d in gat:
                d.wait()
            scat[g] = fire_scatters(g)
        for g_ in sorted(scat):
            for d in scat.pop(g_):
                d.wait()

        plsc.subcore_barrier()
        for q in range(4):
            sl = pl.ds(base + q * STAGE_ROWS, STAGE_ROWS)
            pltpu.sync_copy(acc_sp.at[sl, :], stage_v)
            pltpu.sync_copy(stage_v, acc_hbm.at[cb, sl, :])
        plsc.subcore_barrier()
        return _

    lax.fori_loop(0, NCB // 2, run_pass, None)


def _s2(edge_index, g):
    return pl.kernel(
        _s2_body,
        out_type=jax.ShapeDtypeStruct((NCB, NP, CBW), jnp.float32),
        mesh=_sc_mesh(),
        scratch_types=[
            pltpu.VMEM((3, G, 2, EK), jnp.int32),
            pltpu.VMEM((2, G * EK, CBW), jnp.float32),
            pltpu.VMEM_SHARED((NP, CBW), jnp.float32),
            pltpu.SemaphoreType.DMA,
            pltpu.SemaphoreType.DMA,
            pltpu.SemaphoreType.DMA,
            pltpu.SemaphoreType.DMA,
        ],
        compiler_params=pltpu.CompilerParams(use_tc_tiling_on_sc=False),
    )(edge_index, g)


_PREC = lax.Precision.DEFAULT


def _k1_body(x_ref, deg_ref, we_ref, be_ref, w1_ref, g_ref, dinv_ref):
    deg = deg_ref[0, :] + deg_ref[1, :] + 1.0
    dinv = lax.rsqrt(deg)
    h = jnp.maximum(
        jnp.dot(x_ref[...], we_ref[...], precision=_PREC,
                preferred_element_type=jnp.float32) + be_ref[...], 0.0)
    g = jnp.dot(h, w1_ref[...], precision=_PREC,
                preferred_element_type=jnp.float32) * dinv[:, None]
    dinv_ref[...] = dinv
    for cb in range(NCB):
        g_ref[cb] = g[:, cb * CBW:(cb + 1) * CBW]


def _k1(x_pad, deg2, W_enc, b_enc2d, W1):
    return pl.pallas_call(
        _k1_body,
        grid=(GRID,),
        in_specs=[
            pl.BlockSpec((R, DIN), lambda i: (i, 0)),
            pl.BlockSpec((2, R), lambda i: (0, i)),
            pl.BlockSpec((DIN, H), lambda i: (0, 0)),
            pl.BlockSpec((1, H), lambda i: (0, 0)),
            pl.BlockSpec((H, H), lambda i: (0, 0)),
        ],
        out_specs=[
            pl.BlockSpec((NCB, R, CBW), lambda i: (0, i, 0)),
            pl.BlockSpec((R,), lambda i: (i,)),
        ],
        out_shape=[
            jax.ShapeDtypeStruct((NCB, NP, CBW), jnp.float32),
            jax.ShapeDtypeStruct((NP,), jnp.float32),
        ],
    )(x_pad, deg2, W_enc, b_enc2d, W1)


def _k2_body(acc_ref, dinv_ref, b_ref, w_ref, g_ref):
    acc = jnp.concatenate([acc_ref[cb] for cb in range(NCB)], axis=1)
    dinv = dinv_ref[...]
    h = jnp.maximum(dinv[:, None] * acc + b_ref[...], 0.0)
    g = jnp.dot(h, w_ref[...], precision=_PREC,
                preferred_element_type=jnp.float32) * dinv[:, None]
    for cb in range(NCB):
        g_ref[cb] = g[:, cb * CBW:(cb + 1) * CBW]


def _k2(acc, dinv, b2d, W):
    return pl.pallas_call(
        _k2_body,
        grid=(GRID,),
        in_specs=[
            pl.BlockSpec((NCB, R, CBW), lambda i: (0, i, 0)),
            pl.BlockSpec((R,), lambda i: (i,)),
            pl.BlockSpec((1, H), lambda i: (0, 0)),
            pl.BlockSpec((H, H), lambda i: (0, 0)),
        ],
        out_specs=pl.BlockSpec((NCB, R, CBW), lambda i: (0, i, 0)),
        out_shape=jax.ShapeDtypeStruct((NCB, NP, CBW), jnp.float32),
    )(acc, dinv, b2d, W)


def _k3_body(acc_ref, dinv_ref, b_ref, wc_ref, bc_ref, out_ref):
    acc = jnp.concatenate([acc_ref[cb] for cb in range(NCB)], axis=1)
    dinv = dinv_ref[...]
    h = jnp.maximum(dinv[:, None] * acc + b_ref[...], 0.0)
    out_ref[...] = jnp.dot(h, wc_ref[...], precision=_PREC,
                           preferred_element_type=jnp.float32) + bc_ref[...]


def _k3(acc, dinv, b2d, W_cls, b_cls2d):
    return pl.pallas_call(
        _k3_body,
        grid=(GRID,),
        in_specs=[
            pl.BlockSpec((NCB, R, CBW), lambda i: (0, i, 0)),
            pl.BlockSpec((R,), lambda i: (i,)),
            pl.BlockSpec((1, H), lambda i: (0, 0)),
            pl.BlockSpec((H, V), lambda i: (0, 0)),
            pl.BlockSpec((1, V), lambda i: (0, 0)),
        ],
        out_specs=pl.BlockSpec((R, V), lambda i: (i, 0)),
        out_shape=jax.ShapeDtypeStruct((NP, V), jnp.float32),
    )(acc, dinv, b2d, W_cls, b_cls2d)


def kernel(x, edge_index, W_enc, b_enc, W1, b1, W2, b2, W_cls, b_cls):
    x_pad = jnp.pad(x, ((0, NP - N), (0, 0)))
    npad = EPAD - E
    pad_src = (jnp.arange(npad, dtype=jnp.int32) % N)
    pad_dst = N + (jnp.arange(npad, dtype=jnp.int32) % (NP - N))
    ei3 = (jnp.concatenate(
        [edge_index, jnp.stack([pad_src, pad_dst])], axis=1)
        .reshape(2, ECHUNKS, EK).transpose(1, 0, 2))
    deg2 = _s0(ei3).reshape(2, NP)
    g1, dinv = _k1(x_pad, deg2, W_enc, b_enc.reshape(1, H), W1)
    acc1 = _s2(ei3, g1)
    g2 = _k2(acc1, dinv, b1.reshape(1, H), W2)
    acc2 = _s2(ei3, g2)
    out = _k3(acc2, dinv, b2.reshape(1, H), W_cls, b_cls.reshape(1, V))
    return out[:N]


# speedup vs baseline: 15.0790x; 1.1036x over previous
import functools

import jax
import jax.numpy as jnp
from jax import lax
from jax.experimental import pallas as pl
from jax.experimental.pallas import tpu as pltpu
from jax.experimental.pallas import tpu_sc as plsc

N = 50000
E = 800000
DIN = 21
H = 128
V = 4

NP = 50176
R = 1024
GRID = NP // R
NCB = 8
CBW = H // NCB
TILE_ROWS = NP // 16
STAGE_ROWS = TILE_ROWS // 4
EK = 256
EPAD = 819200
ECHUNKS = EPAD // EK
TILE_CHUNKS = ECHUNKS // 16
G = 8
GROUPS = TILE_CHUNKS // G
S0_TILE_CHUNKS = ECHUNKS // 2 // 16


def _sc_mesh():
    return plsc.VectorSubcoreMesh(core_axis_name="c", subcore_axis_name="s")


def _s0_body(ei_hbm, deg_hbm, idx_v, ones_v, zbuf_v, deg_sp):
    c = lax.axis_index("c")
    s = lax.axis_index("s")

    def fill(buf, n, val):
        def body(i, _):
            buf[pl.ds(i * 16, 16)] = jnp.full((16,), val, jnp.float32)
            return _
        lax.fori_loop(0, n // 16, body, None)

    fill(ones_v, EK, 1.0)
    fill(zbuf_v, TILE_ROWS, 0.0)
    pltpu.sync_copy(zbuf_v, deg_sp.at[pl.ds(s * TILE_ROWS, TILE_ROWS)])
    plsc.subcore_barrier()

    row0 = (c * 16 + s) * S0_TILE_CHUNKS

    def edge_chunk(j, _):
        pltpu.sync_copy(ei_hbm.at[row0 + j, 1, :], idx_v)
        pltpu.sync_copy(ones_v, deg_sp.at[idx_v], add=True)
        return _

    lax.fori_loop(0, S0_TILE_CHUNKS, edge_chunk, None)
    plsc.subcore_barrier()
    sl = pl.ds(s * TILE_ROWS, TILE_ROWS)
    pltpu.sync_copy(deg_sp.at[sl], zbuf_v)
    pltpu.sync_copy(zbuf_v, deg_hbm.at[pl.ds(c * NP + s * TILE_ROWS, TILE_ROWS)])


def _s0(edge_index):
    return pl.kernel(
        _s0_body,
        out_type=jax.ShapeDtypeStruct((2 * NP,), jnp.float32),
        mesh=_sc_mesh(),
        scratch_types=[
            pltpu.VMEM((EK,), jnp.int32),
            pltpu.VMEM((EK,), jnp.float32),
            pltpu.VMEM((TILE_ROWS,), jnp.float32),
            pltpu.VMEM_SHARED((NP,), jnp.float32),
        ],
        compiler_params=pltpu.CompilerParams(use_tc_tiling_on_sc=False),
    )(edge_index)


def _s2_body(ei_hbm, g_hbm, acc_hbm, ibuf, msg_v, acc_sp,
             sem_i, sem_g, sem_s0, sem_s1):
    c = lax.axis_index("c")
    s = lax.axis_index("s")
    base = s * TILE_ROWS
    row_base = s * TILE_CHUNKS
    sem_s = (sem_s0, sem_s1)

    def run_pass(p, _):
        cb = (NCB // 2) * c + p
        stage_v = msg_v.at[0, pl.ds(0, STAGE_ROWS), :]
        for q in range(4):
            sl = pl.ds(base + q * STAGE_ROWS, STAGE_ROWS)
            pltpu.sync_copy(g_hbm.at[cb, sl, :], stage_v)
            pltpu.sync_copy(stage_v, acc_sp.at[sl, :])

        def fire_idx(g):
            return pltpu.async_copy(
                ei_hbm.at[pl.ds(row_base + g * G, G), :, :],
                ibuf.at[g % 3], sem_i)

        def fire_gathers(g):
            ds_ = []
            for i in range(G):
                ds_.append(pltpu.async_copy(
                    g_hbm.at[cb].at[ibuf.at[g % 3, i, 0]],
                    msg_v.at[g % 2, pl.ds(i * EK, EK), :], sem_g))
            return ds_

        def fire_scatters(g):
            ds_ = []
            for i in range(G):
                ds_.append(pltpu.async_copy(
                    msg_v.at[g % 2, pl.ds(i * EK, EK), :],
                    acc_sp.at[ibuf.at[g % 3, i, 1]], sem_s[g % 2],
                    add=True))
            return ds_

        idx_d = fire_idx(0)
        plsc.subcore_barrier()

        scat = {}
        for g in range(GROUPS):
            idx_d.wait()
            if g >= 2:
                for d in scat.pop(g - 2):
                    d.wait()
            gat = fire_gathers(g)
            if g + 1 < GROUPS:
                idx_d = fire_idx(g + 1)
            for d in gat:
                d.wait()
            scat[g] = fire_scatters(g)
        for g_ in sorted(scat):
            for d in scat.pop(g_):
                d.wait()

        plsc.subcore_barrier()
        for q in range(4):
            sl = pl.ds(base + q * STAGE_ROWS, STAGE_ROWS)
            pltpu.sync_copy(acc_sp.at[sl, :], stage_v)
            pltpu.sync_copy(stage_v, acc_hbm.at[cb, sl, :])
        plsc.subcore_barrier()
        return _

    lax.fori_loop(0, NCB // 2, run_pass, None)


def _s2(edge_index, g):
    return pl.kernel(
        _s2_body,
        out_type=jax.ShapeDtypeStruct((NCB, NP, CBW), jnp.float32),
        mesh=_sc_mesh(),
        scratch_types=[
            pltpu.VMEM((3, G, 2, EK), jnp.int32),
            pltpu.VMEM((2, G * EK, CBW), jnp.float32),
            pltpu.VMEM_SHARED((NP, CBW), jnp.float32),
            pltpu.SemaphoreType.DMA,
            pltpu.SemaphoreType.DMA,
            pltpu.SemaphoreType.DMA,
            pltpu.SemaphoreType.DMA,
        ],
        compiler_params=pltpu.CompilerParams(use_tc_tiling_on_sc=False),
    )(edge_index, g)


_PREC = lax.Precision.DEFAULT


def _k1_body(x_ref, deg_ref, we_ref, be_ref, w1_ref, g_ref, dinv_ref):
    deg = deg_ref[0, :] + deg_ref[1, :] + 1.0
    dinv = lax.rsqrt(deg)
    h = jnp.maximum(
        jnp.dot(x_ref[...], we_ref[...], precision=_PREC,
                preferred_element_type=jnp.float32) + be_ref[...], 0.0)
    g = jnp.dot(h, w1_ref[...], precision=_PREC,
                preferred_element_type=jnp.float32) * dinv[:, None]
    dinv_ref[...] = dinv
    for cb in range(NCB):
        g_ref[cb] = g[:, cb * CBW:(cb + 1) * CBW]


def _k1(x_pad, deg2, W_enc, b_enc2d, W1):
    return pl.pallas_call(
        _k1_body,
        grid=(GRID,),
        in_specs=[
            pl.BlockSpec((R, DIN), lambda i: (i, 0)),
            pl.BlockSpec((2, R), lambda i: (0, i)),
            pl.BlockSpec((DIN, H), lambda i: (0, 0)),
            pl.BlockSpec((1, H), lambda i: (0, 0)),
            pl.BlockSpec((H, H), lambda i: (0, 0)),
        ],
        out_specs=[
            pl.BlockSpec((NCB, R, CBW), lambda i: (0, i, 0)),
            pl.BlockSpec((R,), lambda i: (i,)),
        ],
        out_shape=[
            jax.ShapeDtypeStruct((NCB, NP, CBW), jnp.float32),
            jax.ShapeDtypeStruct((NP,), jnp.float32),
        ],
    )(x_pad, deg2, W_enc, b_enc2d, W1)


def _k2_body(acc_ref, dinv_ref, b_ref, w_ref, g_ref):
    acc = jnp.concatenate([acc_ref[cb] for cb in range(NCB)], axis=1)
    dinv = dinv_ref[...]
    h = jnp.maximum(dinv[:, None] * acc + b_ref[...], 0.0)
    g = jnp.dot(h, w_ref[...], precision=_PREC,
                preferred_element_type=jnp.float32) * dinv[:, None]
    for cb in range(NCB):
        g_ref[cb] = g[:, cb * CBW:(cb + 1) * CBW]


def _k2(acc, dinv, b2d, W):
    return pl.pallas_call(
        _k2_body,
        grid=(GRID,),
        in_specs=[
            pl.BlockSpec((NCB, R, CBW), lambda i: (0, i, 0)),
            pl.BlockSpec((R,), lambda i: (i,)),
            pl.BlockSpec((1, H), lambda i: (0, 0)),
            pl.BlockSpec((H, H), lambda i: (0, 0)),
        ],
        out_specs=pl.BlockSpec((NCB, R, CBW), lambda i: (0, i, 0)),
        out_shape=jax.ShapeDtypeStruct((NCB, NP, CBW), jnp.float32),
    )(acc, dinv, b2d, W)


def _k3_body(acc_ref, dinv_ref, b_ref, wc_ref, bc_ref, out_ref):
    acc = jnp.concatenate([acc_ref[cb] for cb in range(NCB)], axis=1)
    dinv = dinv_ref[...]
    h = jnp.maximum(dinv[:, None] * acc + b_ref[...], 0.0)
    out_ref[...] = jnp.dot(h, wc_ref[...], precision=_PREC,
                           preferred_element_type=jnp.float32) + bc_ref[...]


def _k3(acc, dinv, b2d, W_cls, b_cls2d):
    return pl.pallas_call(
        _k3_body,
        grid=(GRID,),
        in_specs=[
            pl.BlockSpec((NCB, R, CBW), lambda i: (0, i, 0)),
            pl.BlockSpec((R,), lambda i: (i,)),
            pl.BlockSpec((1, H), lambda i: (0, 0)),
            pl.BlockSpec((H, V), lambda i: (0, 0)),
            pl.BlockSpec((1, V), lambda i: (0, 0)),
        ],
        out_specs=pl.BlockSpec((R, V), lambda i: (i, 0)),
        out_shape=jax.ShapeDtypeStruct((NP, V), jnp.float32),
    )(acc, dinv, b2d, W_cls, b_cls2d)


def kernel(x, edge_index, W_enc, b_enc, W1, b1, W2, b2, W_cls, b_cls):
    x_pad = jnp.pad(x, ((0, NP - N), (0, 0)))
    npad = EPAD - E
    pad_src = (jnp.arange(npad, dtype=jnp.int32) % N)
    pad_dst = N + (jnp.arange(npad, dtype=jnp.int32) % (NP - N))
    ei3 = (jnp.concatenate(
        [edge_index, jnp.stack([pad_src, pad_dst])], axis=1)
        .reshape(2, ECHUNKS, EK).transpose(1, 0, 2))
    deg2 = _s0(ei3).reshape(2, NP)
    g1, dinv = _k1(x_pad, deg2, W_enc, b_enc.reshape(1, H), W1)
    acc1 = _s2(ei3, g1)
    g2 = _k2(acc1, dinv, b1.reshape(1, H), W2)
    acc2 = _s2(ei3, g2)
    out = _k3(acc2, dinv, b2.reshape(1, H), W_cls, b_cls.reshape(1, V))
    return out[:N]


# speedup vs baseline: 15.3226x; 1.0162x over previous
import functools

import jax
import jax.numpy as jnp
from jax import lax
from jax.experimental import pallas as pl
from jax.experimental.pallas import tpu as pltpu
from jax.experimental.pallas import tpu_sc as plsc

N = 50000
E = 800000
DIN = 21
H = 128
V = 4

NP = 50176
R = 1024
GRID = NP // R
NCB = 8
CBW = H // NCB
TILE_ROWS = NP // 16
STAGE_ROWS = TILE_ROWS // 4
EK = 512
EPAD = 819200
ECHUNKS = EPAD // EK
TILE_CHUNKS = ECHUNKS // 16
G = 4
GROUPS = TILE_CHUNKS // G
S0_TILE_CHUNKS = ECHUNKS // 2 // 16


def _sc_mesh():
    return plsc.VectorSubcoreMesh(core_axis_name="c", subcore_axis_name="s")


def _s0_body(ei_hbm, deg_hbm, idx_v, ones_v, zbuf_v, deg_sp):
    c = lax.axis_index("c")
    s = lax.axis_index("s")

    def fill(buf, n, val):
        def body(i, _):
            buf[pl.ds(i * 16, 16)] = jnp.full((16,), val, jnp.float32)
            return _
        lax.fori_loop(0, n // 16, body, None)

    fill(ones_v, EK, 1.0)
    fill(zbuf_v, TILE_ROWS, 0.0)
    pltpu.sync_copy(zbuf_v, deg_sp.at[pl.ds(s * TILE_ROWS, TILE_ROWS)])
    plsc.subcore_barrier()

    row0 = (c * 16 + s) * S0_TILE_CHUNKS

    def edge_chunk(j, _):
        pltpu.sync_copy(ei_hbm.at[row0 + j, 1, :], idx_v)
        pltpu.sync_copy(ones_v, deg_sp.at[idx_v], add=True)
        return _

    lax.fori_loop(0, S0_TILE_CHUNKS, edge_chunk, None)
    plsc.subcore_barrier()
    sl = pl.ds(s * TILE_ROWS, TILE_ROWS)
    pltpu.sync_copy(deg_sp.at[sl], zbuf_v)
    pltpu.sync_copy(zbuf_v, deg_hbm.at[pl.ds(c * NP + s * TILE_ROWS, TILE_ROWS)])


def _s0(edge_index):
    return pl.kernel(
        _s0_body,
        out_type=jax.ShapeDtypeStruct((2 * NP,), jnp.float32),
        mesh=_sc_mesh(),
        scratch_types=[
            pltpu.VMEM((EK,), jnp.int32),
            pltpu.VMEM((EK,), jnp.float32),
            pltpu.VMEM((TILE_ROWS,), jnp.float32),
            pltpu.VMEM_SHARED((NP,), jnp.float32),
        ],
        compiler_params=pltpu.CompilerParams(use_tc_tiling_on_sc=False),
    )(edge_index)


def _s2_body(ei_hbm, g_hbm, acc_hbm, ibuf, msg_v, acc_sp,
             sem_i, sem_g, sem_s0, sem_s1):
    c = lax.axis_index("c")
    s = lax.axis_index("s")
    base = s * TILE_ROWS
    row_base = s * TILE_CHUNKS
    sem_s = (sem_s0, sem_s1)

    def run_pass(p, _):
        cb = (NCB // 2) * c + p
        stage_v = msg_v.at[0, pl.ds(0, STAGE_ROWS), :]
        for q in range(4):
            sl = pl.ds(base + q * STAGE_ROWS, STAGE_ROWS)
            pltpu.sync_copy(g_hbm.at[cb, sl, :], stage_v)
            pltpu.sync_copy(stage_v, acc_sp.at[sl, :])

        def fire_idx(g):
            return pltpu.async_copy(
                ei_hbm.at[pl.ds(row_base + g * G, G), :, :],
                ibuf.at[g % 3], sem_i)

        def fire_gathers(g):
            ds_ = []
            for i in range(G):
                ds_.append(pltpu.async_copy(
                    g_hbm.at[cb].at[ibuf.at[g % 3, i, 0]],
                    msg_v.at[g % 2, pl.ds(i * EK, EK), :], sem_g))
            return ds_

        def fire_scatters(g):
            ds_ = []
            for i in range(G):
                ds_.append(pltpu.async_copy(
                    msg_v.at[g % 2, pl.ds(i * EK, EK), :],
                    acc_sp.at[ibuf.at[g % 3, i, 1]], sem_s[g % 2],
                    add=True))
            return ds_

        idx_d = fire_idx(0)
        plsc.subcore_barrier()

        scat = {}
        for g in range(GROUPS):
            idx_d.wait()
            if g >= 2:
                for d in scat.pop(g - 2):
                    d.wait()
            gat = fire_gathers(g)
            if g + 1 < GROUPS:
                idx_d = fire_idx(g + 1)
            for d in gat:
                d.wait()
            scat[g] = fire_scatters(g)
        for g_ in sorted(scat):
            for d in scat.pop(g_):
                d.wait()

        plsc.subcore_barrier()
        for q in range(4):
            sl = pl.ds(base + q * STAGE_ROWS, STAGE_ROWS)
            pltpu.sync_copy(acc_sp.at[sl, :], stage_v)
            pltpu.sync_copy(stage_v, acc_hbm.at[cb, sl, :])
        plsc.subcore_barrier()
        return _

    lax.fori_loop(0, NCB // 2, run_pass, None)


def _s2(edge_index, g):
    return pl.kernel(
        _s2_body,
        out_type=jax.ShapeDtypeStruct((NCB, NP, CBW), jnp.float32),
        mesh=_sc_mesh(),
        scratch_types=[
            pltpu.VMEM((3, G, 2, EK), jnp.int32),
            pltpu.VMEM((2, G * EK, CBW), jnp.float32),
            pltpu.VMEM_SHARED((NP, CBW), jnp.float32),
            pltpu.SemaphoreType.DMA,
            pltpu.SemaphoreType.DMA,
            pltpu.SemaphoreType.DMA,
            pltpu.SemaphoreType.DMA,
        ],
        compiler_params=pltpu.CompilerParams(use_tc_tiling_on_sc=False),
    )(edge_index, g)


_PREC = lax.Precision.DEFAULT


def _k1_body(x_ref, deg_ref, we_ref, be_ref, w1_ref, g_ref, dinv_ref):
    deg = deg_ref[0, :] + deg_ref[1, :] + 1.0
    dinv = lax.rsqrt(deg)
    h = jnp.maximum(
        jnp.dot(x_ref[...], we_ref[...], precision=_PREC,
                preferred_element_type=jnp.float32) + be_ref[...], 0.0)
    g = jnp.dot(h, w1_ref[...], precision=_PREC,
                preferred_element_type=jnp.float32) * dinv[:, None]
    dinv_ref[...] = dinv
    for cb in range(NCB):
        g_ref[cb] = g[:, cb * CBW:(cb + 1) * CBW]


def _k1(x_pad, deg2, W_enc, b_enc2d, W1):
    return pl.pallas_call(
        _k1_body,
        grid=(GRID,),
        in_specs=[
            pl.BlockSpec((R, DIN), lambda i: (i, 0)),
            pl.BlockSpec((2, R), lambda i: (0, i)),
            pl.BlockSpec((DIN, H), lambda i: (0, 0)),
            pl.BlockSpec((1, H), lambda i: (0, 0)),
            pl.BlockSpec((H, H), lambda i: (0, 0)),
        ],
        out_specs=[
            pl.BlockSpec((NCB, R, CBW), lambda i: (0, i, 0)),
            pl.BlockSpec((R,), lambda i: (i,)),
        ],
        out_shape=[
            jax.ShapeDtypeStruct((NCB, NP, CBW), jnp.float32),
            jax.ShapeDtypeStruct((NP,), jnp.float32),
        ],
    )(x_pad, deg2, W_enc, b_enc2d, W1)


def _k2_body(acc_ref, dinv_ref, b_ref, w_ref, g_ref):
    acc = jnp.concatenate([acc_ref[cb] for cb in range(NCB)], axis=1)
    dinv = dinv_ref[...]
    h = jnp.maximum(dinv[:, None] * acc + b_ref[...], 0.0)
    g = jnp.dot(h, w_ref[...], precision=_PREC,
                preferred_element_type=jnp.float32) * dinv[:, None]
    for cb in range(NCB):
        g_ref[cb] = g[:, cb * CBW:(cb + 1) * CBW]


def _k2(acc, dinv, b2d, W):
    return pl.pallas_call(
        _k2_body,
        grid=(GRID,),
        in_specs=[
            pl.BlockSpec((NCB, R, CBW), lambda i: (0, i, 0)),
            pl.BlockSpec((R,), lambda i: (i,)),
            pl.BlockSpec((1, H), lambda i: (0, 0)),
            pl.BlockSpec((H, H), lambda i: (0, 0)),
        ],
        out_specs=pl.BlockSpec((NCB, R, CBW), lambda i: (0, i, 0)),
        out_shape=jax.ShapeDtypeStruct((NCB, NP, CBW), jnp.float32),
    )(acc, dinv, b2d, W)


def _k3_body(acc_ref, dinv_ref, b_ref, wc_ref, bc_ref, out_ref):
    acc = jnp.concatenate([acc_ref[cb] for cb in range(NCB)], axis=1)
    dinv = dinv_ref[...]
    h = jnp.maximum(dinv[:, None] * acc + b_ref[...], 0.0)
    out_ref[...] = jnp.dot(h, wc_ref[...], precision=_PREC,
                           preferred_element_type=jnp.float32) + bc_ref[...]


def _k3(acc, dinv, b2d, W_cls, b_cls2d):
    return pl.pallas_call(
        _k3_body,
        grid=(GRID,),
        in_specs=[
            pl.BlockSpec((NCB, R, CBW), lambda i: (0, i, 0)),
            pl.BlockSpec((R,), lambda i: (i,)),
            pl.BlockSpec((1, H), lambda i: (0, 0)),
            pl.BlockSpec((H, V), lambda i: (0, 0)),
            pl.BlockSpec((1, V), lambda i: (0, 0)),
        ],
        out_specs=pl.BlockSpec((R, V), lambda i: (i, 0)),
        out_shape=jax.ShapeDtypeStruct((NP, V), jnp.float32),
    )(acc, dinv, b2d, W_cls, b_cls2d)


def kernel(x, edge_index, W_enc, b_enc, W1, b1, W2, b2, W_cls, b_cls):
    x_pad = jnp.pad(x, ((0, NP - N), (0, 0)))
    npad = EPAD - E
    pad_src = (jnp.arange(npad, dtype=jnp.int32) % N)
    pad_dst = N + (jnp.arange(npad, dtype=jnp.int32) % (NP - N))
    ei3 = (jnp.concatenate(
        [edge_index, jnp.stack([pad_src, pad_dst])], axis=1)
        .reshape(2, ECHUNKS, EK).transpose(1, 0, 2))
    deg2 = _s0(ei3).reshape(2, NP)
    g1, dinv = _k1(x_pad, deg2, W_enc, b_enc.reshape(1, H), W1)
    acc1 = _s2(ei3, g1)
    g2 = _k2(acc1, dinv, b1.reshape(1, H), W2)
    acc2 = _s2(ei3, g2)
    out = _k3(acc2, dinv, b2.reshape(1, H), W_cls, b_cls.reshape(1, V))
    return out[:N]
